# Initial kernel scaffold; baseline (speedup 1.0000x reference)
import jax
import jax.numpy as jnp
from jax.experimental import pallas as pl


def kernel(indices, W_q, W_r):
    raise NotImplementedError("write your pallas kernel here")


# speedup vs baseline: 15.8320x; 15.8320x over previous
import functools

import jax
import jax.numpy as jnp
from jax import lax
from jax.experimental import pallas as pl
from jax.experimental.pallas import tpu as pltpu, tpu_sc as plsc

_BATCH = 16384
_F = 26
_D = 64
_C = 1000
_NW = 32
_BPW = _BATCH // _NW
_BLK = 32
_NBLK = _BPW // _BLK


def _qr_split(v):
    q = (v.astype(jnp.float32) * jnp.float32(1.0 / _C)).astype(jnp.int32)
    r = v - q * _C
    too_big = r >= _C
    too_small = r < 0
    q = jnp.where(too_big, q + 1, jnp.where(too_small, q - 1, q))
    r = jnp.where(too_big, r - _C, jnp.where(too_small, r + _C, r))
    return q, r


def _body(idx_hbm, wq_hbm, wr_hbm, out_hbm,
          idx_v, qidx_v, ridx_v, gq_v, gr_v, outb_v, sem_q, sem_r):
    nc = 2
    wid = lax.axis_index("s") * nc + lax.axis_index("c")
    row0 = wid * _BPW

    @pl.loop(0, _NBLK)
    def _block(blk):
        base = row0 + blk * _BLK
        for f in range(_F):
            pltpu.sync_copy(idx_hbm.at[f, pl.ds(base, _BLK)], idx_v)
            for c in range(_BLK // 16):
                v = idx_v[pl.ds(c * 16, 16)]
                q, r = _qr_split(v)
                off = jnp.full((16,), f * _C, jnp.int32)
                qidx_v[pl.ds(c * 16, 16)] = q + off
                ridx_v[pl.ds(c * 16, 16)] = r + off
            cp_q = pltpu.async_copy(wq_hbm.at[qidx_v], gq_v, sem_q)
            cp_r = pltpu.async_copy(wr_hbm.at[ridx_v], gr_v, sem_r)
            cp_q.wait()
            cp_r.wait()

            @pl.loop(0, _BLK)
            def _row(i):
                for c in range(_D // 16):
                    prod = gq_v[i, pl.ds(c * 16, 16)] * gr_v[i, pl.ds(c * 16, 16)]
                    outb_v[i, pl.ds(f * _D + c * 16, 16)] = prod

        pltpu.sync_copy(outb_v, out_hbm.at[pl.ds(base, _BLK), :])


@jax.jit
def _qr_embedding(idx_t, wq_flat, wr_flat):
    mesh = plsc.VectorSubcoreMesh(core_axis_name="c", subcore_axis_name="s")
    return pl.kernel(
        _body,
        out_type=jax.ShapeDtypeStruct((_BATCH, _F * _D), jnp.float32),
        mesh=mesh,
        compiler_params=pltpu.CompilerParams(use_tc_tiling_on_sc=False),
        scratch_types=[
            pltpu.VMEM((_BLK,), jnp.int32),
            pltpu.VMEM((_BLK,), jnp.int32),
            pltpu.VMEM((_BLK,), jnp.int32),
            pltpu.VMEM((_BLK, _D), jnp.float32),
            pltpu.VMEM((_BLK, _D), jnp.float32),
            pltpu.VMEM((_BLK, _F * _D), jnp.float32),
            pltpu.SemaphoreType.DMA,
            pltpu.SemaphoreType.DMA,
        ],
    )(idx_t, wq_flat, wr_flat)


def kernel(indices, W_q, W_r):
    idx_t = indices.T
    wq_flat = W_q.reshape(_F * 1000, _D)
    wr_flat = W_r.reshape(_F * _C, _D)
    return _qr_embedding(idx_t, wq_flat, wr_flat)


# speedup vs baseline: 48.0434x; 3.0346x over previous
import functools

import jax
import jax.numpy as jnp
from jax import lax
from jax.experimental import pallas as pl
from jax.experimental.pallas import tpu as pltpu, tpu_sc as plsc

_BATCH = 16384
_F = 26
_D = 64
_C = 1000
_NW = 32
_BPW = _BATCH // _NW
_CH = 128
_NCH = _BPW // _CH
_NSTEP = _F * _NCH
_NSLOT = 4


def _qr_split(v):
    q = (v.astype(jnp.float32) * jnp.float32(1.0 / _C)).astype(jnp.int32)
    r = v - q * _C
    too_big = r >= _C
    too_small = r < 0
    q = jnp.where(too_big, q + 1, jnp.where(too_small, q - 1, q))
    r = jnp.where(too_big, r - _C, jnp.where(too_small, r + _C, r))
    return q, r


def _body(idx_hbm, wq_hbm, wr_hbm, out_hbm, idxb, qidx, ridx, gq, gr, *sems):
    semq = sems[0:4]
    semr = sems[4:8]
    semo = sems[8:12]
    wid = lax.axis_index("s") * 2 + lax.axis_index("c")
    row0 = wid * _BPW

    pltpu.sync_copy(idx_hbm.at[:, pl.ds(row0, _BPW)], idxb)

    @pl.loop(0, _F)
    def _prep(f):
        off = jnp.full((16,), f * _C, jnp.int32)
        for ch in range(_NCH):
            for j in range(_CH // 16):
                v = idxb[f, pl.ds(ch * _CH + j * 16, 16)]
                q, r = _qr_split(v)
                qidx[f * _NCH + ch, pl.ds(j * 16, 16)] = q + off
                ridx[f * _NCH + ch, pl.ds(j * 16, 16)] = r + off

    def _fire(s, slot):
        pltpu.async_copy(wq_hbm.at[qidx.at[s]], gq.at[slot], semq[slot])
        pltpu.async_copy(wr_hbm.at[ridx.at[s]], gr.at[slot], semr[slot])

    def _wait_gather(slot):
        pltpu.make_async_copy(wq_hbm.at[pl.ds(0, _CH)], gq.at[slot], semq[slot]).wait()
        pltpu.make_async_copy(wr_hbm.at[pl.ds(0, _CH)], gr.at[slot], semr[slot]).wait()

    def _wait_out(slot):
        pltpu.make_async_copy(
            gq.at[slot], out_hbm.at[pl.ds(0, _CH), pl.ds(0, _D)], semo[slot]
        ).wait()

    for b in range(_NSLOT - 1):
        _fire(b, b)

    @pl.loop(0, _NSTEP, step=_NSLOT)
    def _main(s0):
        f = s0 // _NCH
        for b in range(_NSLOT):
            s3 = s0 + b + (_NSLOT - 1)

            @pl.when(s3 < _NSTEP)
            def _():
                _fire(s3, (b + _NSLOT - 1) % _NSLOT)

            _wait_gather(b)

            gqb = gq.at[b]
            grb = gr.at[b]

            @pl.loop(0, _CH)
            def _mul(i):
                for c in range(_D // 16):
                    gqb[i, pl.ds(c * 16, 16)] = (
                        gqb[i, pl.ds(c * 16, 16)] * grb[i, pl.ds(c * 16, 16)]
                    )

            @pl.when(s0 > 0)
            def _():
                _wait_out(b)

            pltpu.async_copy(
                gq.at[b],
                out_hbm.at[pl.ds(row0 + b * _CH, _CH), pl.ds(f * _D, _D)],
                semo[b],
            )

    for b in range(_NSLOT):
        _wait_out(b)


@jax.jit
def _qr_embedding(idx_t, wq_flat, wr_flat):
    mesh = plsc.VectorSubcoreMesh(core_axis_name="c", subcore_axis_name="s")
    return pl.kernel(
        _body,
        out_type=jax.ShapeDtypeStruct((_BATCH, _F * _D), jnp.float32),
        mesh=mesh,
        compiler_params=pltpu.CompilerParams(use_tc_tiling_on_sc=False),
        scratch_types=[
            pltpu.VMEM((_F, _BPW), jnp.int32),
            pltpu.VMEM((_NSTEP, _CH), jnp.int32),
            pltpu.VMEM((_NSTEP, _CH), jnp.int32),
            pltpu.VMEM((_NSLOT, _CH, _D), jnp.float32),
            pltpu.VMEM((_NSLOT, _CH, _D), jnp.float32),
        ] + [pltpu.SemaphoreType.DMA] * 12,
    )(idx_t, wq_flat, wr_flat)


def kernel(indices, W_q, W_r):
    idx_t = indices.T
    wq_flat = W_q.reshape(_F * _C, _D)
    wr_flat = W_r.reshape(_F * _C, _D)
    return _qr_embedding(idx_t, wq_flat, wr_flat)
